# Initial kernel scaffold; baseline (speedup 1.0000x reference)
import functools

import jax
import jax.numpy as jnp
from jax.experimental import pallas as pl
from jax.experimental.pallas import tpu as pltpu

N_BINS = 15


def _ece_body(block_n, n_total, n_steps,
              logits_ref, labels_ref, out_ref,
              count_s, conf_s, acc_s, maxlab_s):
    i = pl.program_id(0)

    @pl.when(i == 0)
    def _init():
        count_s[...] = jnp.zeros_like(count_s)
        conf_s[...] = jnp.zeros_like(conf_s)
        acc_s[...] = jnp.zeros_like(acc_s)
        maxlab_s[0] = 0

    x = logits_ref[...]
    m = jnp.max(x, axis=1, keepdims=True)
    e = jnp.exp(x - m)
    s = jnp.sum(e, axis=1, keepdims=True)
    p = e / s

    lab = labels_ref[...]
    maxlab_s[0] = jnp.maximum(maxlab_s[0], jnp.max(lab))

    cls = jax.lax.broadcasted_iota(jnp.int32, p.shape, 1)
    is_lab = (cls == lab).astype(jnp.float32)

    bini = jnp.clip(jnp.ceil(p * N_BINS).astype(jnp.int32) - 1, 0, N_BINS - 1)
    bini = jnp.where(p > 0.0, bini, -1)

    for b in range(N_BINS):
        mask = (bini == b).astype(jnp.float32)
        count_s[b, :] += jnp.sum(mask, axis=0)
        conf_s[b, :] += jnp.sum(mask * p, axis=0)
        acc_s[b, :] += jnp.sum(mask * is_lab, axis=0)

    @pl.when(i == n_steps - 1)
    def _fini():
        nc = maxlab_s[0] + 1
        count = count_s[...]
        nonempty = count > 0.0
        denom = jnp.maximum(count, 1.0)
        avg_conf = jnp.where(nonempty, conf_s[...] / denom, 0.0)
        avg_acc = jnp.where(nonempty, acc_s[...] / denom, 0.0)
        prop = count / jnp.float32(n_total)
        contrib = jnp.where(nonempty, jnp.abs(avg_conf - avg_acc) * prop, 0.0)
        cls1 = jax.lax.broadcasted_iota(jnp.int32, count.shape, 1)
        contrib = jnp.where(cls1 < nc, contrib, 0.0)
        out_ref[0, 0] = jnp.sum(contrib) / nc.astype(jnp.float32)


def kernel(logits, labels):
    N, C = logits.shape
    block_n = 1000
    n_steps = N // block_n
    labels2d = labels.reshape(N, 1)

    body = functools.partial(_ece_body, block_n, N, n_steps)
    out = pl.pallas_call(
        body,
        grid=(n_steps,),
        in_specs=[
            pl.BlockSpec((block_n, C), lambda i: (i, 0)),
            pl.BlockSpec((block_n, 1), lambda i: (i, 0)),
        ],
        out_specs=pl.BlockSpec((1, 1), lambda i: (0, 0)),
        out_shape=jax.ShapeDtypeStruct((1, 1), jnp.float32),
        scratch_shapes=[
            pltpu.VMEM((N_BINS, C), jnp.float32),
            pltpu.VMEM((N_BINS, C), jnp.float32),
            pltpu.VMEM((N_BINS, C), jnp.float32),
            pltpu.SMEM((1,), jnp.int32),
        ],
    )(logits, labels2d)
    return out[0, 0]


# speedup vs baseline: 104.2351x; 104.2351x over previous
import functools

import jax
import jax.numpy as jnp
from jax.experimental import pallas as pl
from jax.experimental.pallas import tpu as pltpu

N_BINS = 15


def _ece_body(block_n, n_total, n_steps,
              logits_ref, labels_ref, out_ref,
              count_s, conf_s, acc_s, maxlab_s):
    i = pl.program_id(0)

    @pl.when(i == 0)
    def _init():
        count_s[...] = jnp.zeros_like(count_s)
        conf_s[...] = jnp.zeros_like(conf_s)
        acc_s[...] = jnp.zeros_like(acc_s)
        maxlab_s[0] = 0

    x = logits_ref[...]
    m = jnp.max(x, axis=1, keepdims=True)
    e = jnp.exp(x - m)
    s = jnp.sum(e, axis=1, keepdims=True)
    p = e / s

    lab = labels_ref[...]
    maxlab_s[0] = jnp.maximum(maxlab_s[0], jnp.max(lab))

    cls = jax.lax.broadcasted_iota(jnp.int32, p.shape, 1)
    is_lab = (cls == lab).astype(jnp.float32)

    bini = jnp.clip(jnp.ceil(p * N_BINS).astype(jnp.int32) - 1, 0, N_BINS - 1)
    bini = jnp.where(p > 0.0, bini, -1)

    for b in range(N_BINS):
        mask = (bini == b).astype(jnp.float32)
        count_s[b, :] += jnp.sum(mask, axis=0)
        conf_s[b, :] += jnp.sum(mask * p, axis=0)
        acc_s[b, :] += jnp.sum(mask * is_lab, axis=0)

    @pl.when(i == n_steps - 1)
    def _fini():
        nc = maxlab_s[0] + 1
        count = count_s[...]
        nonempty = count > 0.0
        denom = jnp.maximum(count, 1.0)
        avg_conf = jnp.where(nonempty, conf_s[...] / denom, 0.0)
        avg_acc = jnp.where(nonempty, acc_s[...] / denom, 0.0)
        prop = count / jnp.float32(n_total)
        contrib = jnp.where(nonempty, jnp.abs(avg_conf - avg_acc) * prop, 0.0)
        cls1 = jax.lax.broadcasted_iota(jnp.int32, count.shape, 1)
        contrib = jnp.where(cls1 < nc, contrib, 0.0)
        total = jnp.sum(contrib, axis=(0, 1), keepdims=True)
        out_ref[...] = total / nc.astype(jnp.float32)


def kernel(logits, labels):
    N, C = logits.shape
    block_n = 1000
    n_steps = N // block_n
    labels2d = labels.reshape(N, 1)

    body = functools.partial(_ece_body, block_n, N, n_steps)
    out = pl.pallas_call(
        body,
        grid=(n_steps,),
        in_specs=[
            pl.BlockSpec((block_n, C), lambda i: (i, 0)),
            pl.BlockSpec((block_n, 1), lambda i: (i, 0)),
        ],
        out_specs=pl.BlockSpec((1, 1), lambda i: (0, 0)),
        out_shape=jax.ShapeDtypeStruct((1, 1), jnp.float32),
        scratch_shapes=[
            pltpu.VMEM((N_BINS, C), jnp.float32),
            pltpu.VMEM((N_BINS, C), jnp.float32),
            pltpu.VMEM((N_BINS, C), jnp.float32),
            pltpu.SMEM((1,), jnp.int32),
        ],
    )(logits, labels2d)
    return out[0, 0]
